# Initial kernel scaffold; baseline (speedup 1.0000x reference)
import jax
import jax.numpy as jnp
from jax.experimental import pallas as pl


def kernel(x, edge_index, ptr, batch, W0, b0, Wfc, bfc, W1, b1, W2, b2, W3, b3):
    raise NotImplementedError("write your pallas kernel here")


# speedup vs baseline: 7.2621x; 7.2621x over previous
import jax
import jax.numpy as jnp
from jax import lax
from jax.experimental import pallas as pl
from jax.experimental.pallas import tpu as pltpu
from jax.experimental.pallas import tpu_sc as plsc

N = 10000
E = 320000
B = 20
NPG = 500
K = 10
HID = 256
DW = 128

NC = 2
NT = 16
CH = 80
NPT = 624
EPC_T = E // NT
EPW = E // (NT * NC)
RB = 1000

_mesh = plsc.VectorSubcoreMesh(core_axis_name="c", subcore_axis_name="s")
f32 = jnp.float32


def _rows_copy(mk_src, mk_dst, s):
    base = s * NPT
    pltpu.sync_copy(mk_src(base, NPT), mk_dst(base, NPT))

    @pl.when(s == NT - 1)
    def _():
        tail = NT * NPT
        pltpu.sync_copy(mk_src(tail, N - tail), mk_dst(tail, N - tail))


def _deg_body(dst_hbm, zeros_hbm, ones_hbm, out_hbm, idx_v, ones_v, acc_sh):
    c = lax.axis_index("c")
    s = lax.axis_index("s")
    pltpu.sync_copy(ones_hbm, ones_v)
    _rows_copy(lambda b, n: zeros_hbm.at[pl.ds(b, n)],
               lambda b, n: acc_sh.at[pl.ds(b, n)], s)
    plsc.subcore_barrier()
    w = c * NT + s

    def body(i, carry):
        base = w * EPW + i * CH
        pltpu.sync_copy(dst_hbm.at[pl.ds(base, CH)], idx_v)
        pltpu.sync_copy(ones_v, acc_sh.at[idx_v], add=True)
        return carry

    lax.fori_loop(0, EPW // CH, body, 0)
    plsc.subcore_barrier()
    _rows_copy(lambda b, n: acc_sh.at[pl.ds(b, n)],
               lambda b, n: out_hbm.at[c, pl.ds(b, n)], s)


_deg_call = pl.kernel(
    _deg_body,
    mesh=_mesh,
    out_type=jax.ShapeDtypeStruct((NC, N, DW), f32),
    scratch_types=[
        pltpu.VMEM((CH,), jnp.int32),
        pltpu.VMEM((CH, DW), f32),
        pltpu.VMEM_SHARED((N, DW), f32),
    ],
)


def _agg_es_body(tab_hbm, src_hbm, dst_hbm, zeros_hbm, out_hbm,
                 idx_s, idx_d, rows, acc_sh, sem):
    c = lax.axis_index("c")
    s = lax.axis_index("s")
    _rows_copy(lambda b, n: zeros_hbm.at[pl.ds(b, n)],
               lambda b, n: acc_sh.at[pl.ds(b, n)], s)
    plsc.subcore_barrier()
    w = c * NT + s

    def it(i, carry):
        base = w * EPW + i * CH
        pltpu.sync_copy(src_hbm.at[pl.ds(base, CH)], idx_s)
        pltpu.sync_copy(dst_hbm.at[pl.ds(base, CH)], idx_d)
        pltpu.async_copy(tab_hbm.at[idx_s], rows, sem).wait()
        pltpu.sync_copy(rows, acc_sh.at[idx_d], add=True)
        return carry

    lax.fori_loop(0, EPW // CH, it, 0)
    plsc.subcore_barrier()
    _rows_copy(lambda b, n: acc_sh.at[pl.ds(b, n)],
               lambda b, n: out_hbm.at[c, pl.ds(b, n)], s)


_agg_es_call = pl.kernel(
    _agg_es_body,
    mesh=_mesh,
    out_type=jax.ShapeDtypeStruct((NC, N, DW), f32),
    scratch_types=[
        pltpu.VMEM((CH,), jnp.int32),
        pltpu.VMEM((CH,), jnp.int32),
        pltpu.VMEM((CH, DW), f32),
        pltpu.VMEM_SHARED((N, DW), f32),
        pltpu.SemaphoreType.DMA,
    ],
)


def _agg_cs_body(tabL, tabR, src_hbm, dst_hbm, zeros_hbm, outL, outR,
                 idx_s, idx_d, rows, acc_sh, sem):
    c = lax.axis_index("c")
    s = lax.axis_index("s")

    def half(tab, out):
        _rows_copy(lambda b, n: zeros_hbm.at[pl.ds(b, n)],
                   lambda b, n: acc_sh.at[pl.ds(b, n)], s)
        plsc.subcore_barrier()

        def it(i, carry):
            base = s * EPC_T + i * CH
            pltpu.sync_copy(src_hbm.at[pl.ds(base, CH)], idx_s)
            pltpu.sync_copy(dst_hbm.at[pl.ds(base, CH)], idx_d)
            pltpu.async_copy(tab.at[idx_s], rows, sem).wait()
            pltpu.sync_copy(rows, acc_sh.at[idx_d], add=True)
            return carry

        lax.fori_loop(0, EPC_T // CH, it, 0)
        plsc.subcore_barrier()
        _rows_copy(lambda b, n: acc_sh.at[pl.ds(b, n)],
                   lambda b, n: out.at[pl.ds(b, n)], s)

    @pl.when(c == 0)
    def _():
        half(tabL, outL)

    @pl.when(c == 1)
    def _():
        half(tabR, outR)


_agg_cs_call = pl.kernel(
    _agg_cs_body,
    mesh=_mesh,
    out_type=[jax.ShapeDtypeStruct((N, DW), f32),
              jax.ShapeDtypeStruct((N, DW), f32)],
    scratch_types=[
        pltpu.VMEM((CH,), jnp.int32),
        pltpu.VMEM((CH,), jnp.int32),
        pltpu.VMEM((CH, DW), f32),
        pltpu.VMEM_SHARED((N, DW), f32),
        pltpu.SemaphoreType.DMA,
    ],
)


def _tc1_body(p0, p1, x, W0, dinv_o, z0L_o, z0R_o):
    deg = p0[...] + p1[...] + 1.0
    dinv = lax.rsqrt(deg)
    dinv_o[...] = dinv
    xw0 = jnp.dot(x[...], W0[...], preferred_element_type=f32)
    z0 = xw0 * dinv
    z0L_o[...] = z0[:, :DW]
    z0R_o[...] = z0[:, DW:]


def _tc2_body(sL, sR, zL, zR, dinv, b0, W1, pooled_o, z1L_o, z1R_o):
    dv = dinv[...]
    agg = jnp.concatenate([(sL[...] + zL[...]) * dv,
                           (sR[...] + zR[...]) * dv], axis=1)
    h = jnp.maximum(agg + b0[...], 0.0)
    rows = lax.broadcasted_iota(jnp.int32, (RB, 1), 0)
    neg = jnp.float32(-jnp.inf)
    m0 = jnp.max(jnp.where(rows < NPG, h, neg), axis=0, keepdims=True)
    m1 = jnp.max(jnp.where(rows >= NPG, h, neg), axis=0, keepdims=True)
    pooled_o[...] = jnp.concatenate([m0, m1], axis=0)[None]
    z1 = jnp.dot(h, W1[...], preferred_element_type=f32) * dv
    z1L_o[...] = z1[:, :DW]
    z1R_o[...] = z1[:, DW:]


def _tcg_body(pooled, Wfc, bfc, gi_o):
    gi_o[...] = jnp.dot(pooled[...], Wfc[...], preferred_element_type=f32) + bfc[...]


def _tc3_body(sL, sR, zL, zR, dinv, b1, W1, z2L_o, z2R_o):
    dv = dinv[...]
    agg = jnp.concatenate([(sL[...] + zL[...]) * dv,
                           (sR[...] + zR[...]) * dv], axis=1)
    l1 = jnp.maximum(agg + b1[...], 0.0)
    z2 = jnp.dot(l1, W1[...], preferred_element_type=f32) * dv
    z2L_o[...] = z2[:, :DW]
    z2R_o[...] = z2[:, DW:]


def _tc4_body(sL, sR, zL, zR, dinv, b1, W2, gi3, z3L_o, z3R_o):
    dv = dinv[...]
    agg = jnp.concatenate([(sL[...] + zL[...]) * dv,
                           (sR[...] + zR[...]) * dv], axis=1)
    l2 = jnp.maximum(agg + b1[...], 0.0)
    g0 = gi3[0, 0:1, :]
    g1 = gi3[0, 1:2, :]
    rows = lax.broadcasted_iota(jnp.int32, (RB, 1), 0)
    gcast = jnp.where(rows < NPG, g0, g1)
    concat = jnp.concatenate([l2, gcast], axis=1)
    z3 = jnp.dot(concat, W2[...], preferred_element_type=f32) * dv
    z3L_o[...] = z3[:, :DW]
    z3R_o[...] = z3[:, DW:]


def _tc5_body(sL, sR, zL, zR, dinv, b2, W3, zb_o):
    dv = dinv[...]
    agg = jnp.concatenate([(sL[...] + zL[...]) * dv,
                           (sR[...] + zR[...]) * dv], axis=1)
    c = jnp.maximum(agg + b2[...], 0.0)
    u = jnp.dot(c, W3[...], preferred_element_type=f32)
    zb_o[...] = jnp.broadcast_to(u * dv, (RB, DW))


def _tc6_body(zu, p0, p1, dinv, out_ref):
    lg = dinv[...] * (p0[...] + p1[...] + zu[...])
    iota = lax.broadcasted_iota(jnp.int32, (B, NPG), 1)
    cur = lg
    kth = None
    for _ in range(K):
        m = jnp.max(cur, axis=1, keepdims=True)
        first = jnp.min(jnp.where(cur == m, iota, NPG), axis=1, keepdims=True)
        cur = jnp.where(iota == first, jnp.float32(-jnp.inf), cur)
        kth = m
    out_ref[...] = (lg >= kth).astype(f32)


def _row_spec(w):
    return pl.BlockSpec((RB, w), lambda j: (j, 0))


def _full_spec(shape):
    nd = len(shape)
    return pl.BlockSpec(shape, lambda j: (0,) * nd)


_GRID = (N // RB,)


def _tc1(p0, p1, x, W0):
    return pl.pallas_call(
        _tc1_body,
        grid=_GRID,
        in_specs=[_row_spec(1), _row_spec(1), _row_spec(128),
                  _full_spec((128, HID))],
        out_specs=[_row_spec(1), _row_spec(DW), _row_spec(DW)],
        out_shape=[jax.ShapeDtypeStruct((N, 1), f32),
                   jax.ShapeDtypeStruct((N, DW), f32),
                   jax.ShapeDtypeStruct((N, DW), f32)],
    )(p0, p1, x, W0)


def _tc2(sL, sR, zL, zR, dinv, b0, W1):
    return pl.pallas_call(
        _tc2_body,
        grid=_GRID,
        in_specs=[_row_spec(DW)] * 4 + [_row_spec(1),
                  _full_spec((1, HID)), _full_spec((HID, HID))],
        out_specs=[pl.BlockSpec((1, 2, HID), lambda j: (j, 0, 0)),
                   _row_spec(DW), _row_spec(DW)],
        out_shape=[jax.ShapeDtypeStruct((N // RB, 2, HID), f32),
                   jax.ShapeDtypeStruct((N, DW), f32),
                   jax.ShapeDtypeStruct((N, DW), f32)],
    )(sL, sR, zL, zR, dinv, b0, W1)


def _tcg(pooled, Wfc, bfc):
    return pl.pallas_call(
        _tcg_body,
        out_shape=jax.ShapeDtypeStruct((B, HID), f32),
    )(pooled, Wfc, bfc)


def _tc3(sL, sR, zL, zR, dinv, b1, W1):
    return pl.pallas_call(
        _tc3_body,
        grid=_GRID,
        in_specs=[_row_spec(DW)] * 4 + [_row_spec(1),
                  _full_spec((1, HID)), _full_spec((HID, HID))],
        out_specs=[_row_spec(DW), _row_spec(DW)],
        out_shape=[jax.ShapeDtypeStruct((N, DW), f32),
                   jax.ShapeDtypeStruct((N, DW), f32)],
    )(sL, sR, zL, zR, dinv, b1, W1)


def _tc4(sL, sR, zL, zR, dinv, b1, W2, gi3):
    return pl.pallas_call(
        _tc4_body,
        grid=_GRID,
        in_specs=[_row_spec(DW)] * 4 + [_row_spec(1),
                  _full_spec((1, HID)), _full_spec((2 * HID, HID)),
                  pl.BlockSpec((1, 2, HID), lambda j: (j, 0, 0))],
        out_specs=[_row_spec(DW), _row_spec(DW)],
        out_shape=[jax.ShapeDtypeStruct((N, DW), f32),
                   jax.ShapeDtypeStruct((N, DW), f32)],
    )(sL, sR, zL, zR, dinv, b1, W2, gi3)


def _tc5(sL, sR, zL, zR, dinv, b2, W3):
    return pl.pallas_call(
        _tc5_body,
        grid=_GRID,
        in_specs=[_row_spec(DW)] * 4 + [_row_spec(1),
                  _full_spec((1, HID)), _full_spec((HID, 1))],
        out_specs=[_row_spec(DW)],
        out_shape=[jax.ShapeDtypeStruct((N, DW), f32)],
    )(sL, sR, zL, zR, dinv, b2, W3)[0]


def _tc6(zu, p0, p1, dinv):
    return pl.pallas_call(
        _tc6_body,
        out_shape=jax.ShapeDtypeStruct((B, NPG), f32),
    )(zu, p0, p1, dinv)


def kernel(x, edge_index, ptr, batch, W0, b0, Wfc, bfc, W1, b1, W2, b2, W3, b3):
    del ptr, batch, b3
    src = edge_index[0]
    dst = edge_index[1]
    zerosNW = jnp.zeros((N, DW), f32)
    onesCW = jnp.ones((CH, DW), f32)

    degp = _deg_call(dst, zerosNW, onesCW)
    dinv, z0L, z0R = _tc1(degp[0, :, 0:1], degp[1, :, 0:1], x, W0)
    s0L, s0R = _agg_cs_call(z0L, z0R, src, dst, zerosNW)
    pooled3, z1L, z1R = _tc2(s0L, s0R, z0L, z0R, dinv,
                             b0.reshape(1, HID), W1)
    gi = _tcg(pooled3.reshape(B, HID), Wfc, bfc.reshape(1, HID))
    s1L, s1R = _agg_cs_call(z1L, z1R, src, dst, zerosNW)
    z2L, z2R = _tc3(s1L, s1R, z1L, z1R, dinv, b1.reshape(1, HID), W1)
    s2L, s2R = _agg_cs_call(z2L, z2R, src, dst, zerosNW)
    z3L, z3R = _tc4(s2L, s2R, z2L, z2R, dinv, b1.reshape(1, HID), W2,
                    gi.reshape(N // RB, 2, HID))
    s3L, s3R = _agg_cs_call(z3L, z3R, src, dst, zerosNW)
    zb = _tc5(s3L, s3R, z3L, z3R, dinv, b2.reshape(1, HID), W3)
    p5 = _agg_es_call(zb, src, dst, zerosNW)
    mask = _tc6(zb[:, 0].reshape(B, NPG),
                p5[0, :, 0].reshape(B, NPG),
                p5[1, :, 0].reshape(B, NPG),
                dinv.reshape(B, NPG))
    return mask.reshape(N, 1)
